# Initial kernel scaffold; baseline (speedup 1.0000x reference)
import jax
import jax.numpy as jnp
from jax.experimental import pallas as pl


def kernel(predictions, targets, anchor_points, stride_tensor):
    raise NotImplementedError("write your pallas kernel here")


# speedup vs baseline: 13.0922x; 13.0922x over previous
import functools
import math

import jax
import jax.numpy as jnp
from jax.experimental import pallas as pl
from jax.experimental.pallas import tpu as pltpu

_NC = 80
_RM = 16
_IMG = 1280
_TOPK = 13
_OHEM = 3
_CW, _BW, _DW = 1.0, 2.5, 0.5

_B = 8
_A = 33600
_F = _NC + 4 * (_RM + 1)
_ABLK = 4224
_NBLK = 8
_AT = _ABLK * _NBLK


def _atan_pos(x):
    hi = x > 2.414213562373095
    mid = x > 0.4142135623730950
    xr = jnp.where(hi, -1.0 / (x + 1e-30),
                   jnp.where(mid, (x - 1.0) / (x + 1.0), x))
    z = xr * xr
    poly = (((8.05374449538e-2 * z - 1.38776856032e-1) * z
             + 1.99777106478e-1) * z - 3.33329491539e-1) * z * xr + xr
    y = jnp.where(hi, math.pi / 2, jnp.where(mid, math.pi / 4, 0.0))
    return y + poly


def _gt_boxes(tgt):
    lab = jnp.clip((tgt[:, 0:1] * _NC).astype(jnp.int32), 0, _NC - 1)
    cx = tgt[:, 1:2] * _IMG
    cy = tgt[:, 2:3] * _IMG
    w = tgt[:, 3:4] * 192.0 + 32.0
    h = tgt[:, 4:5] * 192.0 + 32.0
    gx1 = jnp.clip(cx - w * 0.5, 0.0, _IMG)
    gy1 = jnp.clip(cy - h * 0.5, 0.0, _IMG)
    gx2 = jnp.clip(cx + w * 0.5, 0.0, _IMG)
    gy2 = jnp.clip(cy + h * 0.5, 0.0, _IMG)
    return lab, gx1, gy1, gx2, gy2


def _loss_kernel(pred_ref, tgt_ref, apst_ref, out_ref,
                 align_s, iou_s, m_s, cand_s, misc_s, logp_s, nli_s, acc_ref):
    i = pl.program_id(0)
    b = pl.program_id(1)

    @pl.when(jnp.logical_and(i == 0, b == 0))
    def _init():
        acc_ref[0, 0] = 0.0

    lane = jax.lax.broadcasted_iota(jnp.int32, (1, _ABLK), 1)
    valid = (b * _ABLK + lane) < _A
    vf = valid.astype(jnp.float32)

    sl = pl.ds(b * _ABLK, _ABLK)

    zc = jnp.where(valid, pred_ref[0, 0:_NC, :], 0.0)
    p = jnp.clip(jax.nn.sigmoid(zc), 1e-7, 1.0 - 1e-7)
    lp = jnp.log(p)
    l1p = jnp.log(1.0 - p)
    base = -jnp.sum(l1p, axis=0, keepdims=True) * vf

    tgt = tgt_ref[0]
    lab, gx1, gy1, gx2, gy2 = _gt_boxes(tgt)
    onehot = (lab == jax.lax.broadcasted_iota(jnp.int32, (32, _NC), 1)
              ).astype(jnp.float32)
    dims = (((1,), (0,)), ((), ()))
    hi = jax.lax.Precision.HIGHEST
    m_gt = jax.lax.dot_general(onehot, l1p - lp, dims, precision=hi)
    cls_sc = jax.lax.dot_general(onehot, p, dims, precision=hi)

    dist = []
    for s4 in range(4):
        zd = jnp.where(valid, pred_ref[0, _NC + 17 * s4:_NC + 17 * s4 + 17, :],
                       0.0)
        mx = jnp.max(zd, axis=0, keepdims=True)
        e = jnp.exp(zd - mx)
        se = jnp.sum(e, axis=0, keepdims=True)
        logp_s[24 * s4:24 * s4 + 17, sl] = zd - mx - jnp.log(se)
        proj = jax.lax.broadcasted_iota(
            jnp.int32, (17, 1), 0).astype(jnp.float32)
        dist.append(jnp.sum(e * proj, axis=0, keepdims=True) / se)

    ax = jnp.where(valid, apst_ref[0:1, :], 0.0)
    ay = jnp.where(valid, apst_ref[1:2, :], 0.0)
    st = jnp.where(valid, apst_ref[2:3, :], 1.0)
    pb0 = ax - dist[0] * st
    pb1 = ay - dist[1] * st
    pb2 = ax + dist[2] * st
    pb3 = ay + dist[3] * st

    misc_s[0:1, sl] = base
    misc_s[1:2, sl] = pb0
    misc_s[2:3, sl] = pb1
    misc_s[3:4, sl] = pb2
    misc_s[4:5, sl] = pb3
    misc_s[5:6, sl] = ax
    misc_s[6:7, sl] = ay
    misc_s[7:8, sl] = st

    ix1 = jnp.maximum(pb0, gx1)
    iy1 = jnp.maximum(pb1, gy1)
    ix2 = jnp.minimum(pb2, gx2)
    iy2 = jnp.minimum(pb3, gy2)
    inter = jnp.clip(ix2 - ix1, 0.0, None) * jnp.clip(iy2 - iy1, 0.0, None)
    pa = jnp.clip(pb2 - pb0, 0.0, None) * jnp.clip(pb3 - pb1, 0.0, None)
    ga = jnp.clip(gx2 - gx1, 0.0, None) * jnp.clip(gy2 - gy1, 0.0, None)
    iou = inter / (pa + ga - inter + 1e-9)

    in_gt = ((ax > gx1) & (ax < gx2) & (ay > gy1) & (ay < gy2)
             ).astype(jnp.float32)
    align = jnp.sqrt(cls_sc) * iou ** 6 * in_gt + in_gt * 1e-11

    align_s[:, sl] = align * vf
    iou_s[:, sl] = iou * vf
    m_s[:, sl] = m_gt

    @pl.when(b == _NBLK - 1)
    def _phase2():
        cand_s[...] = jnp.zeros((32, _AT), jnp.float32)

        def tal_iter(_, carry):
            v = align_s[...]
            mx = jnp.max(v, axis=1, keepdims=True)
            li = jax.lax.broadcasted_iota(jnp.int32, (32, _AT), 1)
            fi = jnp.min(jnp.where(v == mx, li, _AT), axis=1, keepdims=True)
            pick = li == fi
            cand_s[...] = cand_s[...] + jnp.where(
                pick & (mx > 1e-11), 1.0, 0.0)
            align_s[...] = jnp.where(pick, -1.0, v)
            return carry

        jax.lax.fori_loop(0, _TOPK, tal_iter, 0)

        cand = cand_s[...]
        iou_v = iou_s[...]
        vc = iou_v * cand
        mxg = jnp.max(vc, axis=0, keepdims=True)
        gi = jax.lax.broadcasted_iota(jnp.int32, (32, _AT), 0)
        assigned = jnp.min(jnp.where(vc == mxg, gi, 32), axis=0,
                           keepdims=True)
        oh_a = (gi == assigned).astype(jnp.float32)
        fgf = (jnp.sum(cand, axis=0, keepdims=True) > 0.0
               ).astype(jnp.float32)
        npos = jnp.sum(fgf)
        num_pos = jnp.maximum(npos, 1.0)
        k = jnp.clip((_OHEM * npos).astype(jnp.int32), 1, _A)

        delta = jnp.sum(m_s[...] * oh_a, axis=0, keepdims=True)
        base_v = misc_s[0:1, :]
        anchor_loss = (base_v + fgf * delta) * (1.0 / _NC)
        pos_loss = jnp.sum(anchor_loss * fgf)
        nl = anchor_loss * (1.0 - fgf)
        nli_s[0:1, :] = jax.lax.bitcast_convert_type(nl, jnp.int32)

        def radix_iter(j, t):
            cbit = jax.lax.shift_left(jnp.int32(1), 30 - j)
            cand_t = jax.lax.bitwise_or(t, cbit)
            cnt = jnp.sum((nli_s[0:1, :] >= cand_t).astype(jnp.int32))
            return jnp.where(cnt >= k, cand_t, t)

        thr_bits = jax.lax.fori_loop(0, 31, radix_iter, jnp.int32(0))
        thr = jax.lax.bitcast_convert_type(thr_bits, jnp.float32)
        gt_mask = (nl > thr).astype(jnp.float32)
        cnt_gt = jnp.sum(gt_mask)
        sum_topk = jnp.sum(nl * gt_mask) + (
            k.astype(jnp.float32) - cnt_gt) * thr
        cls_loss = (pos_loss + sum_topk) / num_pos

        tgt2 = tgt_ref[0]
        _, gx1b, gy1b, gx2b, gy2b = _gt_boxes(tgt2)
        tb0 = jnp.sum(oh_a * gx1b, axis=0, keepdims=True)
        tb1 = jnp.sum(oh_a * gy1b, axis=0, keepdims=True)
        tb2 = jnp.sum(oh_a * gx2b, axis=0, keepdims=True)
        tb3 = jnp.sum(oh_a * gy2b, axis=0, keepdims=True)

        pb0 = misc_s[1:2, :]
        pb1 = misc_s[2:3, :]
        pb2 = misc_s[3:4, :]
        pb3 = misc_s[4:5, :]
        axv = misc_s[5:6, :]
        ayv = misc_s[6:7, :]
        stv = misc_s[7:8, :]

        eps = 1e-9
        pw = jnp.clip(pb2 - pb0, 0.0, None)
        ph = jnp.clip(pb3 - pb1, 0.0, None)
        gw = jnp.clip(tb2 - tb0, 0.0, None)
        gh = jnp.clip(tb3 - tb1, 0.0, None)
        ix1 = jnp.maximum(pb0, tb0)
        iy1 = jnp.maximum(pb1, tb1)
        ix2 = jnp.minimum(pb2, tb2)
        iy2 = jnp.minimum(pb3, tb3)
        inter = jnp.clip(ix2 - ix1, 0.0, None) * jnp.clip(iy2 - iy1, 0.0, None)
        union = pw * ph + gw * gh - inter + eps
        iou_b = inter / union
        cx1 = jnp.minimum(pb0, tb0)
        cy1 = jnp.minimum(pb1, tb1)
        cx2 = jnp.maximum(pb2, tb2)
        cy2 = jnp.maximum(pb3, tb3)
        c2 = jnp.clip(cx2 - cx1, 0.0, None) ** 2 + \
            jnp.clip(cy2 - cy1, 0.0, None) ** 2 + eps
        rho2 = ((pb0 + pb2) * 0.5 - (tb0 + tb2) * 0.5) ** 2 + \
            ((pb1 + pb3) * 0.5 - (tb1 + tb3) * 0.5) ** 2
        v4 = 4.0 / math.pi ** 2 * (
            _atan_pos(gw / (gh + eps)) - _atan_pos(pw / (ph + eps))) ** 2
        alpha_v = v4 / (1.0 - iou_b + v4 + eps)
        ciou = 1.0 - iou_b + rho2 / c2 + alpha_v * v4
        box_sum = jnp.sum(ciou * fgf)

        t_sides = (
            jnp.clip((axv - tb0) / stv, 0.0, _RM - 0.01),
            jnp.clip((ayv - tb1) / stv, 0.0, _RM - 0.01),
            jnp.clip((tb2 - axv) / stv, 0.0, _RM - 0.01),
            jnp.clip((tb3 - ayv) / stv, 0.0, _RM - 0.01),
        )
        dfl_acc = jnp.zeros((1, _AT), jnp.float32)
        for s4 in range(4):
            tv = t_sides[s4]
            tl = tv.astype(jnp.int32)
            wl = (tl + 1).astype(jnp.float32) - tv
            wr = 1.0 - wl
            ce = jnp.zeros((1, _AT), jnp.float32)
            for j in range(_RM + 1):
                row = logp_s[24 * s4 + j:24 * s4 + j + 1, :]
                w = jnp.where(tl == j, wl, 0.0)
                if j >= 1:
                    w = w + jnp.where(tl == j - 1, wr, 0.0)
                ce = ce - row * w
            dfl_acc = dfl_acc + ce
        dfl_sum = jnp.sum(dfl_acc * 0.25 * fgf)

        img_loss = _CW * cls_loss + (_BW * box_sum + _DW * dfl_sum) / num_pos
        acc_ref[0, 0] = acc_ref[0, 0] + img_loss / _B

        @pl.when(i == _B - 1)
        def _emit():
            out_ref[...] = jnp.full((1, 1), acc_ref[0, 0], jnp.float32)


@jax.jit
def kernel(predictions, targets, anchor_points, stride_tensor):
    pred_t = jnp.swapaxes(predictions, 1, 2)
    apst = jnp.concatenate(
        [anchor_points.T, stride_tensor.T], axis=0)

    out = pl.pallas_call(
        _loss_kernel,
        grid=(_B, _NBLK),
        in_specs=[
            pl.BlockSpec((1, _F, _ABLK), lambda i, b: (i, 0, b)),
            pl.BlockSpec((1, 32, 5), lambda i, b: (i, 0, 0)),
            pl.BlockSpec((3, _ABLK), lambda i, b: (0, b)),
        ],
        out_specs=pl.BlockSpec((1, 1), lambda i, b: (0, 0)),
        out_shape=jax.ShapeDtypeStruct((1, 1), jnp.float32),
        scratch_shapes=[
            pltpu.VMEM((32, _AT), jnp.float32),
            pltpu.VMEM((32, _AT), jnp.float32),
            pltpu.VMEM((32, _AT), jnp.float32),
            pltpu.VMEM((32, _AT), jnp.float32),
            pltpu.VMEM((8, _AT), jnp.float32),
            pltpu.VMEM((96, _AT), jnp.float32),
            pltpu.VMEM((8, _AT), jnp.int32),
            pltpu.SMEM((1, 1), jnp.float32),
        ],
        compiler_params=pltpu.CompilerParams(
            dimension_semantics=("arbitrary", "arbitrary"),
        ),
    )(pred_t, targets, apst)
    return out[0, 0]
